# baseline (speedup 1.0000x reference)
import functools

import jax
import jax.numpy as jnp
from jax import lax
from jax.experimental import pallas as pl
from jax.experimental.pallas import tpu as pltpu
from jax.experimental.pallas import tpu_sc as plsc

_VOCAB = 100000
_D = 64
_BATCH = 4096
_SEQ = 200

_NC = 2
_NS = 16
_NW = _NC * _NS
_ROWS_W = _BATCH // _NW

_SPLITS = ((0, 104), (104, 96))
_CROWS = 2
_STREAMS = _CROWS * 2
_NCHUNK = _ROWS_W // _CROWS
_NBUF = 2
_SCALE = 8.0


def _gather_body(table_hbm, idx_hbm, out_hbm, idx_v, rows_v, gsems, osems):
    wid = lax.axis_index("s") * _NC + lax.axis_index("c")
    row_base = wid * _ROWS_W

    pltpu.sync_copy(idx_hbm.at[pl.ds(row_base, _ROWS_W)], idx_v)

    def fire_gathers(g, b):
        copies = []
        for r in range(_CROWS):
            for off, ln in _SPLITS:
                copies.append(
                    pltpu.async_copy(
                        table_hbm.at[idx_v.at[g * _CROWS + r].at[pl.ds(off, ln)]],
                        rows_v.at[b, r].at[pl.ds(off, ln)],
                        gsems.at[b]))
        return copies

    def scale_chunk(b):
        for r in range(_CROWS):
            rv = rows_v.at[b, r]

            @plsc.parallel_loop(0, _SEQ, unroll=4)
            def _(i):
                for j in range(_D // 16):
                    s = pl.ds(j * 16, 16)
                    rv[i, s] = rv[i, s] * _SCALE

    out_copies = [None] * _NBUF
    gathers = fire_gathers(0, 0)
    for g in range(_NCHUNK):
        b = g % _NBUF
        nb = (g + 1) % _NBUF
        if g + 1 < _NCHUNK:
            if out_copies[nb] is not None:
                out_copies[nb].wait()
            next_gathers = fire_gathers(g + 1, nb)
        for c in gathers:
            c.wait()
        scale_chunk(b)
        out_copies[b] = pltpu.async_copy(
            rows_v.at[b],
            out_hbm.at[pl.ds(row_base + g * _CROWS, _CROWS)],
            osems.at[b])
        if g + 1 < _NCHUNK:
            gathers = next_gathers
    for c in out_copies:
        if c is not None:
            c.wait()


@jax.jit
def _sc_gather(table, x):
    mesh = plsc.VectorSubcoreMesh(core_axis_name="c", subcore_axis_name="s")
    return pl.kernel(
        _gather_body,
        out_type=jax.ShapeDtypeStruct((_BATCH, _SEQ, _D), jnp.float32),
        mesh=mesh,
        scratch_types=[
            pltpu.VMEM((_ROWS_W, _SEQ), jnp.int32),
            pltpu.VMEM((_NBUF, _CROWS, _SEQ, _D), jnp.float32),
            pltpu.SemaphoreType.DMA((_NBUF,)),
            pltpu.SemaphoreType.DMA((_NBUF,)),
        ],
        compiler_params=pltpu.CompilerParams(use_tc_tiling_on_sc=False),
    )(table, x)


def kernel(x, table):
    return _sc_gather(table, x)
